# Initial kernel scaffold; baseline (speedup 1.0000x reference)
import jax
import jax.numpy as jnp
from jax.experimental import pallas as pl


def kernel(x, edge_index, edge_w, edge_type, tasks, task_emb_table, fc1_W, fc1_b, W_rel, W_self, b_gnn):
    raise NotImplementedError("write your pallas kernel here")


# speedup vs baseline: 6.9086x; 6.9086x over previous
import functools

import jax
import jax.numpy as jnp
from jax import lax
from jax.experimental import pallas as pl
from jax.experimental.pallas import tpu as pltpu
from jax.experimental.pallas import tpu_sc as plsc

N_NODES = 10000
IN_DIM = 128
HID = 256
HALF = 128
NT = 3
E_TOTAL = 320000
CHUNK = 128
NC, NS = 2, 16
N_CHUNKS = E_TOTAL // CHUNK
RBLK = 400
BLK = 1000


def _stage0_body(xin_ref, fc1w_ref, fc1b_ref, wrel_ref, wself_ref, bg_ref,
                 y_ref, s_ref):
    z = jnp.dot(xin_ref[...], fc1w_ref[...],
                preferred_element_type=jnp.float32) + fc1b_ref[...]
    for t in range(NT):
        yt = jnp.dot(z, wrel_ref[t], preferred_element_type=jnp.float32)
        y_ref[0, t] = yt[:, :HALF]
        y_ref[1, t] = yt[:, HALF:]
    s = jnp.dot(z, wself_ref[...], preferred_element_type=jnp.float32) + bg_ref[...]
    s_ref[0] = s[:, :HALF]
    s_ref[1] = s[:, HALF:]


def _stagel_body(o_ref, wrel_ref, wself_ref, bg_ref, y_ref, s_ref):
    z = jnp.concatenate([o_ref[0], o_ref[1]], axis=-1)
    z = jnp.maximum(z, 0.0)
    for t in range(NT):
        yt = jnp.dot(z, wrel_ref[t], preferred_element_type=jnp.float32)
        y_ref[0, t] = yt[:, :HALF]
        y_ref[1, t] = yt[:, HALF:]
    s = jnp.dot(z, wself_ref[...], preferred_element_type=jnp.float32) + bg_ref[...]
    s_ref[0] = s[:, :HALF]
    s_ref[1] = s[:, HALF:]


def _final_body(o_ref, out_ref):
    z = jnp.concatenate([o_ref[0], o_ref[1]], axis=-1)
    out_ref[...] = jnp.maximum(z, 0.0)


_Y_SPEC = pl.BlockSpec((NC, NT, BLK, HALF), lambda i: (0, 0, i, 0))
_S_SPEC = pl.BlockSpec((NC, BLK, HALF), lambda i: (0, i, 0))
_Y_SHAPE = jax.ShapeDtypeStruct((NC, NT, N_NODES, HALF), jnp.float32)
_S_SHAPE = jax.ShapeDtypeStruct((NC, N_NODES, HALF), jnp.float32)


def _stage0(xin, fc1_W, fc1_b, wrel, wself, bg):
    return pl.pallas_call(
        _stage0_body,
        grid=(N_NODES // BLK,),
        in_specs=[
            pl.BlockSpec((BLK, IN_DIM), lambda i: (i, 0)),
            pl.BlockSpec((IN_DIM, HID), lambda i: (0, 0)),
            pl.BlockSpec((1, HID), lambda i: (0, 0)),
            pl.BlockSpec((NT, HID, HID), lambda i: (0, 0, 0)),
            pl.BlockSpec((HID, HID), lambda i: (0, 0)),
            pl.BlockSpec((1, HID), lambda i: (0, 0)),
        ],
        out_specs=[_Y_SPEC, _S_SPEC],
        out_shape=[_Y_SHAPE, _S_SHAPE],
    )(xin, fc1_W, fc1_b, wrel, wself, bg)


def _stagel(o, wrel, wself, bg):
    return pl.pallas_call(
        _stagel_body,
        grid=(N_NODES // BLK,),
        in_specs=[
            pl.BlockSpec((NC, BLK, HALF), lambda i: (0, i, 0)),
            pl.BlockSpec((NT, HID, HID), lambda i: (0, 0, 0)),
            pl.BlockSpec((HID, HID), lambda i: (0, 0)),
            pl.BlockSpec((1, HID), lambda i: (0, 0)),
        ],
        out_specs=[_Y_SPEC, _S_SPEC],
        out_shape=[_Y_SHAPE, _S_SHAPE],
    )(o, wrel, wself, bg)


def _final(o):
    return pl.pallas_call(
        _final_body,
        grid=(N_NODES // BLK,),
        in_specs=[pl.BlockSpec((NC, BLK, HALF), lambda i: (0, i, 0))],
        out_specs=pl.BlockSpec((BLK, HID), lambda i: (i, 0)),
        out_shape=jax.ShapeDtypeStruct((N_NODES, HID), jnp.float32),
    )(o)


def _sc_scatter_kernel(y_hbm, s_hbm, off_hbm, dst_hbm, w_hbm, o_hbm,
                       off_v, dst_v, w_v, rows_v, acc, sem):
    c = lax.axis_index("c")
    s = lax.axis_index("s")
    n_rblk = N_NODES // RBLK

    def init_body(i, carry):
        k = i * NS + s

        @pl.when(k < n_rblk)
        def _():
            pltpu.sync_copy(s_hbm.at[c, pl.ds(k * RBLK, RBLK)],
                            acc.at[pl.ds(k * RBLK, RBLK)])

        return carry

    lax.fori_loop(0, (n_rblk + NS - 1) // NS, init_body, 0)
    plsc.subcore_barrier()

    def chunk_body(i, carry):
        k = i * NS + s

        @pl.when(k < N_CHUNKS)
        def _():
            base = k * CHUNK
            pltpu.sync_copy(off_hbm.at[c, pl.ds(base, CHUNK)], off_v)
            pltpu.sync_copy(dst_hbm.at[pl.ds(base, CHUNK)], dst_v)
            pltpu.sync_copy(w_hbm.at[pl.ds(base, CHUNK)], w_v)
            pltpu.async_copy(y_hbm.at[off_v], rows_v, sem).wait()

            def scale_group(g, carry2):
                wg = w_v[pl.ds(g * 16, 16)]
                for jl in range(16):
                    wv = lax.gather(
                        wg, jnp.full((16, 1), jl, jnp.int32),
                        lax.GatherDimensionNumbers(
                            offset_dims=(), collapsed_slice_dims=(0,),
                            start_index_map=(0,)),
                        slice_sizes=(1,),
                        mode=lax.GatherScatterMode.PROMISE_IN_BOUNDS)
                    j = g * 16 + jl
                    for cc in range(HALF // 16):
                        sl = pl.ds(cc * 16, 16)
                        rows_v[j, sl] = rows_v[j, sl] * wv
                return carry2

            lax.fori_loop(0, CHUNK // 16, scale_group, 0)
            pltpu.sync_copy(rows_v, acc.at[dst_v], add=True)

        return carry

    lax.fori_loop(0, (N_CHUNKS + NS - 1) // NS, chunk_body, 0)
    plsc.subcore_barrier()

    def wb_body(i, carry):
        k = i * NS + s

        @pl.when(k < n_rblk)
        def _():
            pltpu.sync_copy(acc.at[pl.ds(k * RBLK, RBLK)],
                            o_hbm.at[c, pl.ds(k * RBLK, RBLK)])

        return carry

    lax.fori_loop(0, (n_rblk + NS - 1) // NS, wb_body, 0)


@functools.cache
def _sc_scatter_built():
    return pl.kernel(
        _sc_scatter_kernel,
        out_type=jax.ShapeDtypeStruct((NC, N_NODES, HALF), jnp.float32),
        mesh=plsc.VectorSubcoreMesh(core_axis_name="c", subcore_axis_name="s",
                                    num_cores=NC, num_subcores=NS),
        scratch_types=[
            pltpu.VMEM((CHUNK,), jnp.int32),
            pltpu.VMEM((CHUNK,), jnp.int32),
            pltpu.VMEM((CHUNK,), jnp.float32),
            pltpu.VMEM((CHUNK, HALF), jnp.float32),
            pltpu.VMEM_SHARED((N_NODES, HALF), jnp.float32),
            pltpu.SemaphoreType.DMA,
        ],
    )


def _sc_scatter(y_flat, sterm, off2, dst, w):
    return _sc_scatter_built()(y_flat, sterm, off2, dst, w)


def kernel(x, edge_index, edge_w, edge_type, tasks, task_emb_table,
           fc1_W, fc1_b, W_rel, W_self, b_gnn):
    n_task = tasks.shape[0]
    rep = x.shape[0] // n_task
    te = jnp.take(task_emb_table, tasks, axis=0)
    te = jnp.repeat(te, rep, axis=0)
    xin = jnp.concatenate([te[:, None, :], x], axis=1).reshape(-1, IN_DIM)

    src = edge_index[0].astype(jnp.int32)
    dst = edge_index[1].astype(jnp.int32)
    off0 = edge_type.astype(jnp.int32) * N_NODES + src
    off2 = jnp.stack([off0, off0 + NT * N_NODES])
    w = edge_w.astype(jnp.float32)

    fc1_b2 = fc1_b.reshape(1, HID)
    z_cur = None
    o = None
    for l in range(W_rel.shape[0]):
        bg2 = b_gnn[l].reshape(1, HID)
        if l == 0:
            y, sterm = _stage0(xin, fc1_W, fc1_b2, W_rel[0], W_self[0], bg2)
        else:
            y, sterm = _stagel(o, W_rel[l], W_self[l], bg2)
        o = _sc_scatter(y.reshape(NC * NT * N_NODES, HALF), sterm, off2, dst, w)
    z = _final(o)
    return z.reshape(n_task, rep, x.shape[1] + 1, HID)


# speedup vs baseline: 11.0214x; 1.5953x over previous
import functools

import jax
import jax.numpy as jnp
from jax import lax
from jax.experimental import pallas as pl
from jax.experimental.pallas import tpu as pltpu
from jax.experimental.pallas import tpu_sc as plsc

N_NODES = 10000
IN_DIM = 128
HID = 256
HALF = 128
NT = 3
E_TOTAL = 320000
CHUNK = 128
NC, NS = 2, 16
N_CHUNKS = E_TOTAL // CHUNK
RBLK = 400
BLK = 1000


def _stage0_body(xin_ref, fc1w_ref, fc1b_ref, wrel_ref, wself_ref, bg_ref,
                 y_ref, s_ref):
    z = jnp.dot(xin_ref[...], fc1w_ref[...],
                preferred_element_type=jnp.float32) + fc1b_ref[...]
    for t in range(NT):
        yt = jnp.dot(z, wrel_ref[t], preferred_element_type=jnp.float32)
        y_ref[0, t] = yt[:, :HALF]
        y_ref[1, t] = yt[:, HALF:]
    s = jnp.dot(z, wself_ref[...], preferred_element_type=jnp.float32) + bg_ref[...]
    s_ref[0] = s[:, :HALF]
    s_ref[1] = s[:, HALF:]


def _stagel_body(o_ref, wrel_ref, wself_ref, bg_ref, y_ref, s_ref):
    z = jnp.concatenate([o_ref[0], o_ref[1]], axis=-1)
    z = jnp.maximum(z, 0.0)
    for t in range(NT):
        yt = jnp.dot(z, wrel_ref[t], preferred_element_type=jnp.float32)
        y_ref[0, t] = yt[:, :HALF]
        y_ref[1, t] = yt[:, HALF:]
    s = jnp.dot(z, wself_ref[...], preferred_element_type=jnp.float32) + bg_ref[...]
    s_ref[0] = s[:, :HALF]
    s_ref[1] = s[:, HALF:]


def _final_body(o_ref, out_ref):
    z = jnp.concatenate([o_ref[0], o_ref[1]], axis=-1)
    out_ref[...] = jnp.maximum(z, 0.0)


_Y_SPEC = pl.BlockSpec((NC, NT, BLK, HALF), lambda i: (0, 0, i, 0))
_S_SPEC = pl.BlockSpec((NC, BLK, HALF), lambda i: (0, i, 0))
_Y_SHAPE = jax.ShapeDtypeStruct((NC, NT, N_NODES, HALF), jnp.float32)
_S_SHAPE = jax.ShapeDtypeStruct((NC, N_NODES, HALF), jnp.float32)


def _stage0(xin, fc1_W, fc1_b, wrel, wself, bg):
    return pl.pallas_call(
        _stage0_body,
        grid=(N_NODES // BLK,),
        in_specs=[
            pl.BlockSpec((BLK, IN_DIM), lambda i: (i, 0)),
            pl.BlockSpec((IN_DIM, HID), lambda i: (0, 0)),
            pl.BlockSpec((1, HID), lambda i: (0, 0)),
            pl.BlockSpec((NT, HID, HID), lambda i: (0, 0, 0)),
            pl.BlockSpec((HID, HID), lambda i: (0, 0)),
            pl.BlockSpec((1, HID), lambda i: (0, 0)),
        ],
        out_specs=[_Y_SPEC, _S_SPEC],
        out_shape=[_Y_SHAPE, _S_SHAPE],
    )(xin, fc1_W, fc1_b, wrel, wself, bg)


def _stagel(o, wrel, wself, bg):
    return pl.pallas_call(
        _stagel_body,
        grid=(N_NODES // BLK,),
        in_specs=[
            pl.BlockSpec((NC, BLK, HALF), lambda i: (0, i, 0)),
            pl.BlockSpec((NT, HID, HID), lambda i: (0, 0, 0)),
            pl.BlockSpec((HID, HID), lambda i: (0, 0)),
            pl.BlockSpec((1, HID), lambda i: (0, 0)),
        ],
        out_specs=[_Y_SPEC, _S_SPEC],
        out_shape=[_Y_SHAPE, _S_SHAPE],
    )(o, wrel, wself, bg)


def _final(o):
    return pl.pallas_call(
        _final_body,
        grid=(N_NODES // BLK,),
        in_specs=[pl.BlockSpec((NC, BLK, HALF), lambda i: (0, i, 0))],
        out_specs=pl.BlockSpec((BLK, HID), lambda i: (i, 0)),
        out_shape=jax.ShapeDtypeStruct((N_NODES, HID), jnp.float32),
    )(o)


def _sc_scatter_kernel(y_hbm, s_hbm, ed_hbm, w_hbm, o_hbm,
                       idx0, idx1, w0, w1, rows0, rows1, acc, sem0, sem1):
    c = lax.axis_index("c")
    s = lax.axis_index("s")
    idx_b = (idx0, idx1)
    w_b = (w0, w1)
    rows_b = (rows0, rows1)
    sem_b = (sem0, sem1)
    n_rblk = N_NODES // RBLK

    def init_body(i, carry):
        k = i * NS + s

        @pl.when(k < n_rblk)
        def _():
            pltpu.sync_copy(s_hbm.at[c, pl.ds(k * RBLK, RBLK)],
                            acc.at[pl.ds(k * RBLK, RBLK)])

        return carry

    lax.fori_loop(0, (n_rblk + NS - 1) // NS, init_body, 0)
    plsc.subcore_barrier()

    n_i = (N_CHUNKS + NS - 1) // NS

    def start(i, b):
        k = i * NS + s
        pltpu.sync_copy(ed_hbm.at[c, k], idx_b[b])
        pltpu.sync_copy(w_hbm.at[pl.ds(k * CHUNK, CHUNK)], w_b[b])
        pltpu.async_copy(y_hbm.at[idx_b[b].at[0]], rows_b[b], sem_b[b])

    def finish(b):
        pltpu.make_async_copy(y_hbm.at[idx_b[b].at[0]], rows_b[b],
                              sem_b[b]).wait()

        def scale_group(g, carry2):
            wg = w_b[b][pl.ds(g * 16, 16)]
            for jl in range(16):
                wv = lax.gather(
                    wg, jnp.full((16, 1), jl, jnp.int32),
                    lax.GatherDimensionNumbers(
                        offset_dims=(), collapsed_slice_dims=(0,),
                        start_index_map=(0,)),
                    slice_sizes=(1,),
                    mode=lax.GatherScatterMode.PROMISE_IN_BOUNDS)
                j = g * 16 + jl
                for cc in range(HALF // 16):
                    sl = pl.ds(cc * 16, 16)
                    rows_b[b][j, sl] = rows_b[b][j, sl] * wv
            return carry2

        lax.fori_loop(0, CHUNK // 16, scale_group, 0)
        pltpu.sync_copy(rows_b[b], acc.at[idx_b[b].at[1]], add=True)

    @pl.when(0 * NS + s < N_CHUNKS)
    def _():
        start(0, 0)

    def pair_body(ii, carry):
        for boff in range(2):
            i = ii * 2 + boff
            b = boff

            @pl.when((i + 1) * NS + s < N_CHUNKS)
            def _():
                start(i + 1, b ^ 1)

            @pl.when(i * NS + s < N_CHUNKS)
            def _():
                finish(b)
        return carry

    lax.fori_loop(0, (n_i + 1) // 2, pair_body, 0)
    plsc.subcore_barrier()

    def wb_body(i, carry):
        k = i * NS + s

        @pl.when(k < n_rblk)
        def _():
            pltpu.sync_copy(acc.at[pl.ds(k * RBLK, RBLK)],
                            o_hbm.at[c, pl.ds(k * RBLK, RBLK)])

        return carry

    lax.fori_loop(0, (n_rblk + NS - 1) // NS, wb_body, 0)


@functools.cache
def _sc_scatter_built():
    return pl.kernel(
        _sc_scatter_kernel,
        out_type=jax.ShapeDtypeStruct((NC, N_NODES, HALF), jnp.float32),
        mesh=plsc.VectorSubcoreMesh(core_axis_name="c", subcore_axis_name="s",
                                    num_cores=NC, num_subcores=NS),
        scratch_types=[
            pltpu.VMEM((2, CHUNK), jnp.int32),
            pltpu.VMEM((2, CHUNK), jnp.int32),
            pltpu.VMEM((CHUNK,), jnp.float32),
            pltpu.VMEM((CHUNK,), jnp.float32),
            pltpu.VMEM((CHUNK, HALF), jnp.float32),
            pltpu.VMEM((CHUNK, HALF), jnp.float32),
            pltpu.VMEM_SHARED((N_NODES, HALF), jnp.float32),
            pltpu.SemaphoreType.DMA,
            pltpu.SemaphoreType.DMA,
        ],
    )


def _sc_scatter(y_flat, sterm, edata, w):
    return _sc_scatter_built()(y_flat, sterm, edata, w)


def kernel(x, edge_index, edge_w, edge_type, tasks, task_emb_table,
           fc1_W, fc1_b, W_rel, W_self, b_gnn):
    n_task = tasks.shape[0]
    rep = x.shape[0] // n_task
    te = jnp.take(task_emb_table, tasks, axis=0)
    te = jnp.repeat(te, rep, axis=0)
    xin = jnp.concatenate([te[:, None, :], x], axis=1).reshape(-1, IN_DIM)

    src = edge_index[0].astype(jnp.int32)
    dst = edge_index[1].astype(jnp.int32)
    off0 = edge_type.astype(jnp.int32) * N_NODES + src
    edata = jnp.stack([
        jnp.stack([off0, dst]),
        jnp.stack([off0 + NT * N_NODES, dst]),
    ]).reshape(NC, 2, N_CHUNKS, CHUNK).transpose(0, 2, 1, 3)
    w = edge_w.astype(jnp.float32)

    fc1_b2 = fc1_b.reshape(1, HID)
    z_cur = None
    o = None
    for l in range(W_rel.shape[0]):
        bg2 = b_gnn[l].reshape(1, HID)
        if l == 0:
            y, sterm = _stage0(xin, fc1_W, fc1_b2, W_rel[0], W_self[0], bg2)
        else:
            y, sterm = _stagel(o, W_rel[l], W_self[l], bg2)
        o = _sc_scatter(y.reshape(NC * NT * N_NODES, HALF), sterm, edata, w)
    z = _final(o)
    return z.reshape(n_task, rep, x.shape[1] + 1, HID)


# speedup vs baseline: 11.0479x; 1.0024x over previous
import functools

import jax
import jax.numpy as jnp
from jax import lax
from jax.experimental import pallas as pl
from jax.experimental.pallas import tpu as pltpu
from jax.experimental.pallas import tpu_sc as plsc

N_NODES = 10000
IN_DIM = 128
HID = 256
HALF = 128
NT = 3
E_TOTAL = 320000
CHUNK = 128
NC, NS = 2, 16
N_CHUNKS = E_TOTAL // CHUNK
RBLK = 400
BLK = 1000


def _stage0_body(xin_ref, fc1w_ref, fc1b_ref, wrel_ref, wself_ref, bg_ref,
                 y_ref, s_ref):
    z = jnp.dot(xin_ref[...], fc1w_ref[...],
                preferred_element_type=jnp.float32) + fc1b_ref[...]
    for t in range(NT):
        yt = jnp.dot(z, wrel_ref[t], preferred_element_type=jnp.float32)
        y_ref[0, t] = yt[:, :HALF]
        y_ref[1, t] = yt[:, HALF:]
    s = jnp.dot(z, wself_ref[...], preferred_element_type=jnp.float32) + bg_ref[...]
    s_ref[0] = s[:, :HALF]
    s_ref[1] = s[:, HALF:]


def _stagel_body(o_ref, wrel_ref, wself_ref, bg_ref, y_ref, s_ref):
    z = jnp.concatenate([o_ref[0], o_ref[1]], axis=-1)
    z = jnp.maximum(z, 0.0)
    for t in range(NT):
        yt = jnp.dot(z, wrel_ref[t], preferred_element_type=jnp.float32)
        y_ref[0, t] = yt[:, :HALF]
        y_ref[1, t] = yt[:, HALF:]
    s = jnp.dot(z, wself_ref[...], preferred_element_type=jnp.float32) + bg_ref[...]
    s_ref[0] = s[:, :HALF]
    s_ref[1] = s[:, HALF:]


def _final_body(o_ref, out_ref):
    z = jnp.concatenate([o_ref[0], o_ref[1]], axis=-1)
    out_ref[...] = jnp.maximum(z, 0.0)


_Y_SPEC = pl.BlockSpec((NC, NT, BLK, HALF), lambda i: (0, 0, i, 0))
_S_SPEC = pl.BlockSpec((NC, BLK, HALF), lambda i: (0, i, 0))
_Y_SHAPE = jax.ShapeDtypeStruct((NC, NT, N_NODES, HALF), jnp.float32)
_S_SHAPE = jax.ShapeDtypeStruct((NC, N_NODES, HALF), jnp.float32)


def _stage0(xin, fc1_W, fc1_b, wrel, wself, bg):
    return pl.pallas_call(
        _stage0_body,
        grid=(N_NODES // BLK,),
        in_specs=[
            pl.BlockSpec((BLK, IN_DIM), lambda i: (i, 0)),
            pl.BlockSpec((IN_DIM, HID), lambda i: (0, 0)),
            pl.BlockSpec((1, HID), lambda i: (0, 0)),
            pl.BlockSpec((NT, HID, HID), lambda i: (0, 0, 0)),
            pl.BlockSpec((HID, HID), lambda i: (0, 0)),
            pl.BlockSpec((1, HID), lambda i: (0, 0)),
        ],
        out_specs=[_Y_SPEC, _S_SPEC],
        out_shape=[_Y_SHAPE, _S_SHAPE],
    )(xin, fc1_W, fc1_b, wrel, wself, bg)


def _stagel(o, wrel, wself, bg):
    return pl.pallas_call(
        _stagel_body,
        grid=(N_NODES // BLK,),
        in_specs=[
            pl.BlockSpec((NC, BLK, HALF), lambda i: (0, i, 0)),
            pl.BlockSpec((NT, HID, HID), lambda i: (0, 0, 0)),
            pl.BlockSpec((HID, HID), lambda i: (0, 0)),
            pl.BlockSpec((1, HID), lambda i: (0, 0)),
        ],
        out_specs=[_Y_SPEC, _S_SPEC],
        out_shape=[_Y_SHAPE, _S_SHAPE],
    )(o, wrel, wself, bg)


def _final(o):
    return pl.pallas_call(
        _final_body,
        grid=(N_NODES // BLK,),
        in_specs=[pl.BlockSpec((NC, BLK, HALF), lambda i: (0, i, 0))],
        out_specs=pl.BlockSpec((BLK, HID), lambda i: (i, 0)),
        out_shape=jax.ShapeDtypeStruct((N_NODES, HID), jnp.float32),
    )(o)


def _sc_scatter_kernel(y_hbm, s_hbm, ed_hbm, w_hbm, o_hbm,
                       idx0, idx1, w0, w1, rows0, rows1, acc,
                       sem0, sem1, ssem0, ssem1):
    c = lax.axis_index("c")
    s = lax.axis_index("s")
    idx_b = (idx0, idx1)
    w_b = (w0, w1)
    rows_b = (rows0, rows1)
    sem_b = (sem0, sem1)
    ssem_b = (ssem0, ssem1)
    n_rblk = N_NODES // RBLK

    def init_body(i, carry):
        k = i * NS + s

        @pl.when(k < n_rblk)
        def _():
            pltpu.sync_copy(s_hbm.at[c, pl.ds(k * RBLK, RBLK)],
                            acc.at[pl.ds(k * RBLK, RBLK)])

        return carry

    lax.fori_loop(0, (n_rblk + NS - 1) // NS, init_body, 0)
    plsc.subcore_barrier()

    n_i = (N_CHUNKS + NS - 1) // NS

    def start(i, b):
        k = i * NS + s
        pltpu.sync_copy(ed_hbm.at[c, k], idx_b[b])
        pltpu.sync_copy(w_hbm.at[pl.ds(k * CHUNK, CHUNK)], w_b[b])
        pltpu.async_copy(y_hbm.at[idx_b[b].at[0]], rows_b[b], sem_b[b])

    def finish(b):
        pltpu.make_async_copy(y_hbm.at[idx_b[b].at[0]], rows_b[b],
                              sem_b[b]).wait()

        def scale_group(g, carry2):
            wg = w_b[b][pl.ds(g * 16, 16)]
            for jl in range(16):
                wv = lax.gather(
                    wg, jnp.full((16, 1), jl, jnp.int32),
                    lax.GatherDimensionNumbers(
                        offset_dims=(), collapsed_slice_dims=(0,),
                        start_index_map=(0,)),
                    slice_sizes=(1,),
                    mode=lax.GatherScatterMode.PROMISE_IN_BOUNDS)
                j = g * 16 + jl
                for cc in range(HALF // 16):
                    sl = pl.ds(cc * 16, 16)
                    rows_b[b][j, sl] = rows_b[b][j, sl] * wv
            return carry2

        lax.fori_loop(0, CHUNK // 16, scale_group, 0)
        pltpu.async_copy(rows_b[b], acc.at[idx_b[b].at[1]], ssem_b[b],
                         add=True)

    def wait_scatter(b):
        pltpu.make_async_copy(rows_b[b], acc.at[idx_b[b].at[1]],
                              ssem_b[b]).wait()

    @pl.when(0 * NS + s < N_CHUNKS)
    def _():
        start(0, 0)

    def pair_body(ii, carry):
        for boff in range(2):
            i = ii * 2 + boff
            b = boff

            @pl.when((i + 1) * NS + s < N_CHUNKS)
            def _():
                @pl.when(i >= 1)
                def _():
                    wait_scatter(b ^ 1)

                start(i + 1, b ^ 1)

            @pl.when(i * NS + s < N_CHUNKS)
            def _():
                finish(b)
        return carry

    lax.fori_loop(0, (n_i + 1) // 2, pair_body, 0)
    wait_scatter(0)
    wait_scatter(1)
    plsc.subcore_barrier()

    def wb_body(i, carry):
        k = i * NS + s

        @pl.when(k < n_rblk)
        def _():
            pltpu.sync_copy(acc.at[pl.ds(k * RBLK, RBLK)],
                            o_hbm.at[c, pl.ds(k * RBLK, RBLK)])

        return carry

    lax.fori_loop(0, (n_rblk + NS - 1) // NS, wb_body, 0)


@functools.cache
def _sc_scatter_built():
    return pl.kernel(
        _sc_scatter_kernel,
        out_type=jax.ShapeDtypeStruct((NC, N_NODES, HALF), jnp.float32),
        mesh=plsc.VectorSubcoreMesh(core_axis_name="c", subcore_axis_name="s",
                                    num_cores=NC, num_subcores=NS),
        scratch_types=[
            pltpu.VMEM((2, CHUNK), jnp.int32),
            pltpu.VMEM((2, CHUNK), jnp.int32),
            pltpu.VMEM((CHUNK,), jnp.float32),
            pltpu.VMEM((CHUNK,), jnp.float32),
            pltpu.VMEM((CHUNK, HALF), jnp.float32),
            pltpu.VMEM((CHUNK, HALF), jnp.float32),
            pltpu.VMEM_SHARED((N_NODES, HALF), jnp.float32),
            pltpu.SemaphoreType.DMA,
            pltpu.SemaphoreType.DMA,
            pltpu.SemaphoreType.DMA,
            pltpu.SemaphoreType.DMA,
        ],
    )


def _sc_scatter(y_flat, sterm, edata, w):
    return _sc_scatter_built()(y_flat, sterm, edata, w)


def kernel(x, edge_index, edge_w, edge_type, tasks, task_emb_table,
           fc1_W, fc1_b, W_rel, W_self, b_gnn):
    n_task = tasks.shape[0]
    rep = x.shape[0] // n_task
    te = jnp.take(task_emb_table, tasks, axis=0)
    te = jnp.repeat(te, rep, axis=0)
    xin = jnp.concatenate([te[:, None, :], x], axis=1).reshape(-1, IN_DIM)

    src = edge_index[0].astype(jnp.int32)
    dst = edge_index[1].astype(jnp.int32)
    off0 = edge_type.astype(jnp.int32) * N_NODES + src
    edata = jnp.stack([
        jnp.stack([off0, dst]),
        jnp.stack([off0 + NT * N_NODES, dst]),
    ]).reshape(NC, 2, N_CHUNKS, CHUNK).transpose(0, 2, 1, 3)
    w = edge_w.astype(jnp.float32)

    fc1_b2 = fc1_b.reshape(1, HID)
    z_cur = None
    o = None
    for l in range(W_rel.shape[0]):
        bg2 = b_gnn[l].reshape(1, HID)
        if l == 0:
            y, sterm = _stage0(xin, fc1_W, fc1_b2, W_rel[0], W_self[0], bg2)
        else:
            y, sterm = _stagel(o, W_rel[l], W_self[l], bg2)
        o = _sc_scatter(y.reshape(NC * NT * N_NODES, HALF), sterm, edata, w)
    z = _final(o)
    return z.reshape(n_task, rep, x.shape[1] + 1, HID)
